# baseline (speedup 1.0000x reference)
import jax
import jax.numpy as jnp
from jax.experimental import pallas as pl

_DILATION = 2


def _copy_kernel(x_ref, o_ref):
    o_ref[...] = x_ref[:, : o_ref.shape[1]]


def kernel(edge_index):
    two, n, kd = edge_index.shape
    k = kd // _DILATION
    rows = 2000
    cols = two * n * kd // rows
    x = edge_index.reshape(rows, cols)
    blk = 400
    out = pl.pallas_call(
        _copy_kernel,
        grid=(rows // blk,),
        in_specs=[pl.BlockSpec((blk, cols), lambda i: (i, 0))],
        out_specs=pl.BlockSpec((blk, cols // _DILATION), lambda i: (i, 0)),
        out_shape=jax.ShapeDtypeStruct((rows, cols // _DILATION), edge_index.dtype),
    )(x)
    return out.reshape(two, n, k)


# speedup vs baseline: 7.8614x; 7.8614x over previous
import functools

import jax
import jax.numpy as jnp
from jax import lax
from jax.experimental import pallas as pl
from jax.experimental.pallas import tpu as pltpu
from jax.experimental.pallas import tpu_sc as plsc

_DILATION = 2


def _sc_plane_gather(x_hbm, o_hbm, buf, sem_in, sem_out):
    c = lax.axis_index("c")
    s = lax.axis_index("s")
    nk = o_hbm.shape[0]

    @pl.when(s == 0)
    def _():
        for idx in range((nk + 1) // 2):
            j = c + 2 * idx
            slot = idx % 2

            @pl.when(j < nk)
            def _():
                cp_in = pltpu.make_async_copy(
                    x_hbm.at[_DILATION * j], buf.at[slot], sem_in.at[slot])
                cp_in.start()
                cp_in.wait()
                cp_out = pltpu.make_async_copy(
                    buf.at[slot], o_hbm.at[j], sem_out.at[slot])
                cp_out.start()
                cp_out.wait()


def kernel(edge_index):
    two, n, kd = edge_index.shape
    k = kd // _DILATION
    xt = jnp.transpose(edge_index, (2, 0, 1))
    mesh = plsc.VectorSubcoreMesh(core_axis_name="c", subcore_axis_name="s")
    run = functools.partial(
        pl.kernel,
        mesh=mesh,
        out_type=jax.ShapeDtypeStruct((k, two, n), edge_index.dtype),
        scratch_types=[
            pltpu.VMEM_SHARED((2, two, n), jnp.int32),
            pltpu.SemaphoreType.DMA((2,)),
            pltpu.SemaphoreType.DMA((2,)),
        ],
    )(_sc_plane_gather)
    out_t = run(xt)
    return jnp.transpose(out_t, (1, 2, 0))


# speedup vs baseline: 9.9445x; 1.2650x over previous
import functools

import jax
import jax.numpy as jnp
from jax import lax
from jax.experimental import pallas as pl
from jax.experimental.pallas import tpu as pltpu
from jax.experimental.pallas import tpu_sc as plsc

_DILATION = 2


def _sc_plane_gather(x_hbm, o_hbm, buf, sem_in, sem_out):
    c = lax.axis_index("c")
    s = lax.axis_index("s")
    nk = o_hbm.shape[0]

    nslot = (nk + 1) // 2

    @pl.when(s == 0)
    def _():
        for idx in range(nslot):
            j = c + 2 * idx

            @pl.when(j < nk)
            def _():
                pltpu.make_async_copy(
                    x_hbm.at[_DILATION * j], buf.at[idx], sem_in.at[idx]
                ).start()

        for idx in range(nslot):
            j = c + 2 * idx

            @pl.when(j < nk)
            def _():
                pltpu.make_async_copy(
                    x_hbm.at[_DILATION * j], buf.at[idx], sem_in.at[idx]
                ).wait()
                pltpu.make_async_copy(
                    buf.at[idx], o_hbm.at[j], sem_out.at[idx]
                ).start()

        for idx in range(nslot):
            j = c + 2 * idx

            @pl.when(j < nk)
            def _():
                pltpu.make_async_copy(
                    buf.at[idx], o_hbm.at[j], sem_out.at[idx]
                ).wait()


def kernel(edge_index):
    two, n, kd = edge_index.shape
    k = kd // _DILATION
    xt = jnp.transpose(edge_index, (2, 0, 1))
    mesh = plsc.VectorSubcoreMesh(core_axis_name="c", subcore_axis_name="s")
    run = functools.partial(
        pl.kernel,
        mesh=mesh,
        out_type=jax.ShapeDtypeStruct((k, two, n), edge_index.dtype),
        scratch_types=[
            pltpu.VMEM_SHARED((5, two, n), jnp.int32),
            pltpu.SemaphoreType.DMA((5,)),
            pltpu.SemaphoreType.DMA((5,)),
        ],
    )(_sc_plane_gather)
    out_t = run(xt)
    return jnp.transpose(out_t, (1, 2, 0))
